# Initial kernel scaffold; baseline (speedup 1.0000x reference)
import jax
import jax.numpy as jnp
from jax.experimental import pallas as pl


def kernel(input_ids, emb_table, W, b):
    raise NotImplementedError("write your pallas kernel here")


# speedup vs baseline: 1.0743x; 1.0743x over previous
import functools

import jax
import jax.numpy as jnp
from jax import lax
from jax.experimental import pallas as pl
from jax.experimental.pallas import tpu as pltpu
from jax.experimental.pallas import tpu_sc as plsc

B = 1024
S = 200
H = 64
V = 100000

NC = 2
NS = 16
NW = NC * NS
BPW = B // NW
CHUNK = 100
CPAD = 104
NCHUNK = S // CHUNK
INV_S = 1.0 / S


def _sc_pool(ids2, emb_table):
    mesh = plsc.VectorSubcoreMesh(core_axis_name="c", subcore_axis_name="s")
    cpw = BPW * NCHUNK

    @functools.partial(
        pl.kernel,
        out_type=jax.ShapeDtypeStruct((B, H), jnp.float32),
        mesh=mesh,
        scratch_types=[
            pltpu.VMEM((cpw, CPAD), jnp.int32),
            pltpu.VMEM((2, NCHUNK, CPAD, H), jnp.float32),
            pltpu.VMEM((BPW, H), jnp.float32),
            pltpu.SemaphoreType.DMA,
            pltpu.SemaphoreType.DMA,
        ],
        compiler_params=pltpu.CompilerParams(use_tc_tiling_on_sc=False),
    )
    def pool(ids_hbm, table_hbm, x_hbm, idx_v, rows_v, out_v, sem0, sem1):
        wid = lax.axis_index("s") * NC + lax.axis_index("c")
        pltpu.sync_copy(ids_hbm.at[pl.ds(wid * cpw, cpw), :], idx_v)
        sems = (sem0, sem1)

        def fire(r, buf):
            return [
                pltpu.async_copy(
                    table_hbm.at[idx_v.at[NCHUNK * r + c]],
                    rows_v.at[buf, c],
                    sems[buf],
                )
                for c in range(NCHUNK)
            ]

        def accum_store(r, buf):
            def sbody(s, accs):
                a = list(accs)
                for u in range(2):
                    s2 = 2 * s + u
                    for c in range(NCHUNK):
                        for g in range(H // 16):
                            a[g] = a[g] + rows_v[buf, c, s2, pl.ds(16 * g, 16)]
                return tuple(a)

            zero = jnp.zeros((16,), jnp.float32)
            accs = lax.fori_loop(0, CHUNK // 2, sbody, (zero,) * (H // 16))
            for g in range(H // 16):
                out_v[r, pl.ds(16 * g, 16)] = accs[g] * INV_S

        pending = {0: fire(0, 0)}
        for r in range(BPW):
            buf = r % 2
            if r + 1 < BPW:
                pending[r + 1] = fire(r + 1, 1 - buf)
            for d in pending.pop(r):
                d.wait()
            accum_store(r, buf)

        pltpu.sync_copy(out_v, x_hbm.at[pl.ds(wid * BPW, BPW), :])

    return pool(ids2, emb_table)


TILE_V = 512


def _mm_body(x_ref, w_ref, b_ref, o_ref):
    o_ref[...] = (
        jnp.dot(x_ref[...], w_ref[...], preferred_element_type=jnp.float32)
        + b_ref[...]
    )


def _tc_project(x, W, b2):
    grid = (pl.cdiv(V, TILE_V),)
    return pl.pallas_call(
        _mm_body,
        grid=grid,
        in_specs=[
            pl.BlockSpec((B, H), lambda i: (0, 0)),
            pl.BlockSpec((H, TILE_V), lambda i: (0, i)),
            pl.BlockSpec((1, TILE_V), lambda i: (0, i)),
        ],
        out_specs=pl.BlockSpec((B, TILE_V), lambda i: (0, i)),
        out_shape=jax.ShapeDtypeStruct((B, V), jnp.float32),
    )(x, W, b2)


def kernel(input_ids, emb_table, W, b):
    ids2 = input_ids.astype(jnp.int32).reshape(B * NCHUNK, CHUNK)
    ids2 = jnp.pad(ids2, ((0, 0), (0, CPAD - CHUNK)))
    x = _sc_pool(ids2, emb_table)
    return _tc_project(x, W, b.reshape(1, V))


# speedup vs baseline: 1.7162x; 1.5975x over previous
import functools

import jax
import jax.numpy as jnp
from jax import lax
from jax.experimental import pallas as pl
from jax.experimental.pallas import tpu as pltpu
from jax.experimental.pallas import tpu_sc as plsc

B = 1024
S = 200
H = 64
V = 100000

NC = 2
NS = 16
NW = NC * NS
BPW = B // NW
CHUNK = 100
CPAD = 104
NCHUNK = S // CHUNK
NBUF = 4
INV_S = 1.0 / S


def _sc_pool(ids2, emb_table):
    mesh = plsc.VectorSubcoreMesh(core_axis_name="c", subcore_axis_name="s")
    cpw = BPW * NCHUNK

    @functools.partial(
        pl.kernel,
        out_type=jax.ShapeDtypeStruct((B, H), jnp.float32),
        mesh=mesh,
        scratch_types=[
            pltpu.VMEM((cpw, CPAD), jnp.int32),
            pltpu.VMEM((NBUF, NCHUNK, CPAD, H), jnp.float32),
            pltpu.VMEM((BPW, H), jnp.float32),
            pltpu.SemaphoreType.DMA,
            pltpu.SemaphoreType.DMA,
            pltpu.SemaphoreType.DMA,
            pltpu.SemaphoreType.DMA,
        ],
        compiler_params=pltpu.CompilerParams(use_tc_tiling_on_sc=False),
    )
    def pool(ids_hbm, table_hbm, x_hbm, idx_v, rows_v, out_v, s0, s1, s2, s3):
        wid = lax.axis_index("s") * NC + lax.axis_index("c")
        pltpu.sync_copy(ids_hbm.at[pl.ds(wid * cpw, cpw), :], idx_v)
        sems = (s0, s1, s2, s3)

        def fire(r, buf):
            return [
                pltpu.async_copy(
                    table_hbm.at[idx_v.at[NCHUNK * r + c]],
                    rows_v.at[buf, c],
                    sems[buf],
                )
                for c in range(NCHUNK)
            ]

        def accum_store(r, buf):
            def sbody(s, accs):
                a = list(accs)
                for u in range(2):
                    s2 = 2 * s + u
                    for c in range(NCHUNK):
                        for g in range(H // 16):
                            a[g] = a[g] + rows_v[buf, c, s2, pl.ds(16 * g, 16)]
                return tuple(a)

            zero = jnp.zeros((16,), jnp.float32)
            accs = lax.fori_loop(0, CHUNK // 2, sbody, (zero,) * (H // 16))
            for g in range(H // 16):
                out_v[r, pl.ds(16 * g, 16)] = accs[g] * INV_S

        pending = {r: fire(r, r) for r in range(NBUF - 1)}
        for r in range(BPW):
            buf = r % NBUF
            if r + NBUF - 1 < BPW:
                pending[r + NBUF - 1] = fire(r + NBUF - 1, (r + NBUF - 1) % NBUF)
            for d in pending.pop(r):
                d.wait()
            accum_store(r, buf)

        pltpu.sync_copy(out_v, x_hbm.at[pl.ds(wid * BPW, BPW), :])

    return pool(ids2, emb_table)


TILE_V = 512


def _mm_body(w_ref, x_ref, b_ref, o_ref):
    o_ref[...] = (
        lax.dot_general(
            w_ref[...], x_ref[...], (((0,), (1,)), ((), ())),
            preferred_element_type=jnp.float32,
        )
        + b_ref[...]
    )


def _tc_project(x, W, b2):
    grid = (pl.cdiv(V, TILE_V),)
    out = pl.pallas_call(
        _mm_body,
        grid=grid,
        in_specs=[
            pl.BlockSpec((H, TILE_V), lambda i: (0, i)),
            pl.BlockSpec((B, H), lambda i: (0, 0)),
            pl.BlockSpec((TILE_V, 1), lambda i: (i, 0)),
        ],
        out_specs=pl.BlockSpec((TILE_V, B), lambda i: (i, 0)),
        out_shape=jax.ShapeDtypeStruct((V, B), jnp.float32),
    )(W, x, b2)
    return out.T


def kernel(input_ids, emb_table, W, b):
    ids2 = input_ids.astype(jnp.int32).reshape(B * NCHUNK, CHUNK)
    ids2 = jnp.pad(ids2, ((0, 0), (0, CPAD - CHUNK)))
    x = _sc_pool(ids2, emb_table)
    return _tc_project(x, W, b.reshape(V, 1))


# speedup vs baseline: 1.9408x; 1.1309x over previous
import functools

import jax
import jax.numpy as jnp
from jax import lax
from jax.experimental import pallas as pl
from jax.experimental.pallas import tpu as pltpu
from jax.experimental.pallas import tpu_sc as plsc

B = 1024
S = 200
H = 64
V = 100000

NC = 2
NS = 16
NW = NC * NS
BPW = B // NW
CHUNK = 100
CPAD = 104
NCHUNK = S // CHUNK
NBUF = 4
INV_S = 1.0 / S


def _sc_pool(ids2, emb_table):
    mesh = plsc.VectorSubcoreMesh(core_axis_name="c", subcore_axis_name="s")
    cpw = BPW * NCHUNK

    @functools.partial(
        pl.kernel,
        out_type=jax.ShapeDtypeStruct((B, H), jnp.float32),
        mesh=mesh,
        scratch_types=[
            pltpu.VMEM((cpw, CPAD), jnp.int32),
            pltpu.VMEM((NBUF, NCHUNK, CPAD, H), jnp.float32),
            pltpu.VMEM((BPW, H), jnp.float32),
            pltpu.SemaphoreType.DMA,
            pltpu.SemaphoreType.DMA,
            pltpu.SemaphoreType.DMA,
            pltpu.SemaphoreType.DMA,
        ],
        compiler_params=pltpu.CompilerParams(use_tc_tiling_on_sc=False),
    )
    def pool(ids_hbm, table_hbm, x_hbm, idx_v, rows_v, out_v, s0, s1, s2, s3):
        wid = lax.axis_index("s") * NC + lax.axis_index("c")
        pltpu.sync_copy(ids_hbm.at[pl.ds(wid * cpw, cpw), :], idx_v)
        sems = (s0, s1, s2, s3)

        def fire(r, buf):
            return [
                pltpu.async_copy(
                    table_hbm.at[idx_v.at[NCHUNK * r + c]],
                    rows_v.at[buf, c],
                    sems[buf],
                )
                for c in range(NCHUNK)
            ]

        def accum_store(r, buf):
            zero = jnp.zeros((16,), jnp.float32)
            ngrp = H // 16

            @plsc.parallel_loop(0, CHUNK, 1, unroll=4,
                                carry=(zero,) * (NCHUNK * ngrp))
            def accs(s, a):
                a = list(a)
                for c in range(NCHUNK):
                    for g in range(ngrp):
                        k = c * ngrp + g
                        a[k] = a[k] + rows_v[buf, c, s, pl.ds(16 * g, 16)]
                return tuple(a)

            for g in range(ngrp):
                tot = accs[g]
                for c in range(1, NCHUNK):
                    tot = tot + accs[c * ngrp + g]
                out_v[r, pl.ds(16 * g, 16)] = tot * INV_S

        pending = {r: fire(r, r) for r in range(NBUF - 1)}
        for r in range(BPW):
            buf = r % NBUF
            if r + NBUF - 1 < BPW:
                pending[r + NBUF - 1] = fire(r + NBUF - 1, (r + NBUF - 1) % NBUF)
            for d in pending.pop(r):
                d.wait()
            accum_store(r, buf)

        pltpu.sync_copy(out_v, x_hbm.at[pl.ds(wid * BPW, BPW), :])

    return pool(ids2, emb_table)


TILE_V = 1024


def _mm_body(w_ref, x_ref, b_ref, o_ref):
    o_ref[...] = (
        lax.dot_general(
            w_ref[...], x_ref[...], (((0,), (1,)), ((), ())),
            preferred_element_type=jnp.float32,
        )
        + b_ref[...]
    )


def _tc_project(x, W, b2):
    grid = (pl.cdiv(V, TILE_V),)
    out = pl.pallas_call(
        _mm_body,
        grid=grid,
        in_specs=[
            pl.BlockSpec((H, TILE_V), lambda i: (0, i)),
            pl.BlockSpec((B, H), lambda i: (0, 0)),
            pl.BlockSpec((TILE_V, 1), lambda i: (i, 0)),
        ],
        out_specs=pl.BlockSpec((TILE_V, B), lambda i: (i, 0)),
        out_shape=jax.ShapeDtypeStruct((V, B), jnp.float32),
    )(W, x, b2)
    return out.T


def kernel(input_ids, emb_table, W, b):
    ids2 = input_ids.astype(jnp.int32).reshape(B * NCHUNK, CHUNK)
    ids2 = jnp.pad(ids2, ((0, 0), (0, CPAD - CHUNK)))
    x = _sc_pool(ids2, emb_table)
    return _tc_project(x, W, b.reshape(V, 1))
